# baseline (speedup 1.0000x reference)
import functools

import jax
import jax.numpy as jnp
from jax import lax
from jax.experimental import pallas as pl
from jax.experimental.pallas import tpu as pltpu
from jax.experimental.pallas import tpu_sc as plsc

_MARGIN_M = 1.0
_MARGIN_T = 1.0

_W = 2048
_SC_M = 16
_CW = 2048

_SC_NC = 2
_SC_NS = 16
_SC_NW = _SC_NC * _SC_NS
_NEG = -3.4e38


def _sc_extract_lab(labels_v, g, rr):
    lslice = labels_v[pl.ds((g // 2) * 16, 16)]
    lane = (g % 2) * 8 + rr
    return jnp.max(jnp.where(lax.iota(jnp.int32, 16) == lane,
                             lslice.astype(jnp.float32), -1.0)
                   ).astype(jnp.int32)


def _sc_process_chunk(buf, labels_v, t, mc, accs, fys):
    g = t // mc
    ch = t % mc
    first = ch == 0
    col0 = ch * _CW

    accs = [jnp.where(first, jnp.full((16,), _NEG, jnp.float32), a)
            for a in accs]
    fys = [jnp.where(first, _NEG, f) for f in fys]

    lanes16 = lax.iota(jnp.int32, 16)
    for rr in range(8):
        lab = _sc_extract_lab(labels_v, g, rr)
        in_c = jnp.logical_and(lab >= col0, lab < col0 + _CW)
        off = jnp.where(in_c, lab - col0, 0)
        sbase = (off // 16) * 16
        lane = off - sbase
        sl = buf[rr, pl.ds(sbase, 16)]
        mask = jnp.logical_and(lanes16 == lane, in_c)
        fy_c = jnp.max(jnp.where(mask, sl, _NEG))
        fys[rr] = jnp.maximum(fys[rr], jnp.where(in_c, fy_c, _NEG))
        buf[rr, pl.ds(sbase, 16)] = jnp.where(mask, -1e10, sl)

    def step(i, carry):
        out = list(carry)
        base = i * 128
        for rr in range(8):
            for j in range(8):
                v = buf[rr, pl.ds(base + j * 16, 16)]
                out[rr] = jnp.maximum(out[rr], v)
        return tuple(out)

    accs = lax.fori_loop(0, _CW // 128, step, tuple(accs))
    return list(accs), fys


def _sc_finalize_group(t, mc, n_chunks, accs, fys, resf_v, resy_v):
    g = t // mc
    last = (t % mc) == mc - 1

    @pl.when(last)
    def _fin():
        lanes16 = lax.iota(jnp.int32, 16)
        vecf = jnp.full((16,), _NEG, jnp.float32)
        vecy = jnp.full((16,), _NEG, jnp.float32)
        for rr in range(8):
            vecf = jnp.where(lanes16 == rr, jnp.max(accs[rr]), vecf)
            vecy = jnp.where(lanes16 == rr, fys[rr], vecy)
        idx = lanes16 + g * 8
        m8 = lanes16 < 8
        plsc.store_scatter(resf_v, [idx], vecf, mask=m8)
        plsc.store_scatter(resy_v, [idx], vecy, mask=m8)


def _sc_worker(rpw, ncls, m, label_hbm, pred_hbm, outf_hbm, outy_hbm,
               labels_v, buf0, buf1, resf_v, resy_v, sem0, sem1):
    cid = lax.axis_index("c")
    sid = lax.axis_index("s")
    wid = sid * _SC_NC + cid
    base_row = wid * rpw
    mc = (m * _W) // _CW
    n_chunks = (rpw // 8) * mc

    def start(t, buf, sem):
        g = t // mc
        ch = t % mc
        pltpu.make_async_copy(
            pred_hbm.at[pl.ds(base_row + g * 8, 8), pl.ds(ch * _CW, _CW)],
            buf, sem).start()

    def wait(t, buf, sem):
        g = t // mc
        ch = t % mc
        pltpu.make_async_copy(
            pred_hbm.at[pl.ds(base_row + g * 8, 8), pl.ds(ch * _CW, _CW)],
            buf, sem).wait()

    pltpu.sync_copy(label_hbm.at[pl.ds(base_row, rpw)], labels_v)
    start(0, buf0, sem0)

    def pair_body(u, carry):
        accs, fys = list(carry[0]), list(carry[1])
        t0 = 2 * u
        start(t0 + 1, buf1, sem1)
        wait(t0, buf0, sem0)
        accs, fys = _sc_process_chunk(buf0, labels_v, t0, mc, accs, fys)

        @pl.when(t0 + 2 < n_chunks)
        def _next():
            start(t0 + 2, buf0, sem0)

        _sc_finalize_group(t0, mc, n_chunks, accs, fys, resf_v, resy_v)

        t1 = t0 + 1
        wait(t1, buf1, sem1)
        accs, fys = _sc_process_chunk(buf1, labels_v, t1, mc, accs, fys)
        _sc_finalize_group(t1, mc, n_chunks, accs, fys, resf_v, resy_v)
        return tuple(accs), tuple(fys)

    init = (tuple(jnp.full((16,), _NEG, jnp.float32) for _ in range(8)),
            tuple(jnp.float32(_NEG) for _ in range(8)))
    lax.fori_loop(0, n_chunks // 2, pair_body, init)

    pltpu.sync_copy(resf_v, outf_hbm.at[pl.ds(base_row, rpw)])
    pltpu.sync_copy(resy_v, outy_hbm.at[pl.ds(base_row, rpw)])


def _sc_loss(prediction, label, m):
    nrows, ncls = prediction.shape
    rpw = nrows // _SC_NW

    mesh = plsc.VectorSubcoreMesh(core_axis_name="c", subcore_axis_name="s")
    call = functools.partial(
        pl.kernel,
        out_type=(jax.ShapeDtypeStruct((nrows,), jnp.float32),
                  jax.ShapeDtypeStruct((nrows,), jnp.float32)),
        mesh=mesh,
        scratch_types=[
            pltpu.VMEM((rpw,), jnp.int32),
            pltpu.VMEM((8, _CW), jnp.float32),
            pltpu.VMEM((8, _CW), jnp.float32),
            pltpu.VMEM((rpw,), jnp.float32),
            pltpu.VMEM((rpw,), jnp.float32),
            pltpu.SemaphoreType.DMA,
            pltpu.SemaphoreType.DMA,
        ],
        compiler_params=pltpu.CompilerParams(needs_layout_passes=False),
    )(functools.partial(_sc_worker, rpw, ncls, m))

    return call(label, prediction)


def _lane_tree_max(v, width):
    while width > 128:
        width //= 2
        v = jnp.maximum(v[:, :width], v[:, width:2 * width])
    return v


def _tc_body(br, ncls, m, jn, label_ref, fsc_ref, ysc_ref, pred_ref,
             out_ref, accm_ref, accy_ref):
    j = pl.program_id(0)

    @pl.when(j == 0)
    def _init():
        accm_ref[...] = jnp.full((br, 128), _NEG, jnp.float32)
        accy_ref[...] = jnp.full((br, 128), _NEG, jnp.float32)

    x = pred_ref[...]
    lab = label_ref[...]
    base = jax.lax.broadcasted_iota(jnp.int32, (br, _W), 1) + (m + j) * _W
    matched = base == lab
    invalid = base >= ncls
    xm = jnp.where(matched | invalid, -1e10, x)
    fyv = jnp.where(matched, x, _NEG)
    accm_ref[...] = jnp.maximum(accm_ref[...], _lane_tree_max(xm, _W))
    accy_ref[...] = jnp.maximum(accy_ref[...], _lane_tree_max(fyv, _W))

    @pl.when(j == jn - 1)
    def _fin():
        fnym = jnp.maximum(jnp.max(accm_ref[...], axis=1, keepdims=True),
                           fsc_ref[...])
        fy = jnp.maximum(jnp.max(accy_ref[...], axis=1, keepdims=True),
                         ysc_ref[...])
        l = (jnp.maximum(_MARGIN_M + _MARGIN_T - fy, 0.0)
             + jnp.maximum(_MARGIN_M + fnym, 0.0))
        out_ref[0, 0] = jnp.sum(l) / br


def _tc_loss(prediction, label, fsc, ysc, m):
    nrows, ncls = prediction.shape
    br = nrows
    jn = pl.cdiv(ncls, _W) - m
    label2 = label.reshape(nrows, 1)

    body = functools.partial(_tc_body, br, ncls, m, jn)
    return pl.pallas_call(
        body,
        grid=(jn,),
        in_specs=[
            pl.BlockSpec((br, 1), lambda j: (0, 0)),
            pl.BlockSpec((br, 1), lambda j: (0, 0)),
            pl.BlockSpec((br, 1), lambda j: (0, 0)),
            pl.BlockSpec((br, _W), lambda j: (0, j + m)),
        ],
        out_specs=pl.BlockSpec((1, 1), lambda j: (0, 0),
                               memory_space=pltpu.SMEM),
        out_shape=jax.ShapeDtypeStruct((1, 1), jnp.float32),
        scratch_shapes=[
            pltpu.VMEM((br, 128), jnp.float32),
            pltpu.VMEM((br, 128), jnp.float32),
        ],
        compiler_params=pltpu.CompilerParams(
            dimension_semantics=("arbitrary",)),
    )(label2, fsc.reshape(nrows, 1), ysc.reshape(nrows, 1), prediction)


def kernel(prediction, label):
    nrows, _ = prediction.shape
    fsc, ysc = _sc_loss(prediction, label, _SC_M)
    out = _tc_loss(prediction, label, fsc, ysc, _SC_M)
    return out[0, 0]


# speedup vs baseline: 1.0619x; 1.0619x over previous
import functools

import jax
import jax.numpy as jnp
from jax import lax
from jax.experimental import pallas as pl
from jax.experimental.pallas import tpu as pltpu
from jax.experimental.pallas import tpu_sc as plsc

_MARGIN_M = 1.0
_MARGIN_T = 1.0

_W = 2048
_SC_M = 16
_CW = 4096

_SC_NC = 2
_SC_NS = 16
_SC_NW = _SC_NC * _SC_NS
_NEG = -3.4e38


def _sc_extract_lab(labels_v, g, rr):
    lslice = labels_v[pl.ds((g // 2) * 16, 16)]
    lane = (g % 2) * 8 + rr
    return jnp.max(jnp.where(lax.iota(jnp.int32, 16) == lane,
                             lslice.astype(jnp.float32), -1.0)
                   ).astype(jnp.int32)


def _sc_process_chunk(buf, labels_v, t, mc, accs, fys):
    g = t // mc
    ch = t % mc
    first = ch == 0
    col0 = ch * _CW

    accs = [jnp.where(first, jnp.full((16,), _NEG, jnp.float32), a)
            for a in accs]
    fys = [jnp.where(first, _NEG, f) for f in fys]

    lanes16 = lax.iota(jnp.int32, 16)
    for rr in range(8):
        lab = _sc_extract_lab(labels_v, g, rr)
        in_c = jnp.logical_and(lab >= col0, lab < col0 + _CW)
        off = jnp.where(in_c, lab - col0, 0)
        sbase = (off // 16) * 16
        lane = off - sbase
        sl = buf[rr, pl.ds(sbase, 16)]
        mask = jnp.logical_and(lanes16 == lane, in_c)
        fy_c = jnp.max(jnp.where(mask, sl, _NEG))
        fys[rr] = jnp.maximum(fys[rr], jnp.where(in_c, fy_c, _NEG))
        buf[rr, pl.ds(sbase, 16)] = jnp.where(mask, -1e10, sl)

    def step(i, carry):
        out = list(carry)
        base = i * 128
        for rr in range(8):
            for j in range(8):
                v = buf[rr, pl.ds(base + j * 16, 16)]
                out[rr] = jnp.maximum(out[rr], v)
        return tuple(out)

    accs = lax.fori_loop(0, _CW // 128, step, tuple(accs))
    return list(accs), fys


def _sc_finalize_group(t, mc, n_chunks, accs, fys, resf_v, resy_v):
    g = t // mc
    last = (t % mc) == mc - 1

    @pl.when(last)
    def _fin():
        lanes16 = lax.iota(jnp.int32, 16)
        vecf = jnp.full((16,), _NEG, jnp.float32)
        vecy = jnp.full((16,), _NEG, jnp.float32)
        for rr in range(8):
            vecf = jnp.where(lanes16 == rr, jnp.max(accs[rr]), vecf)
            vecy = jnp.where(lanes16 == rr, fys[rr], vecy)
        idx = lanes16 + g * 8
        m8 = lanes16 < 8
        plsc.store_scatter(resf_v, [idx], vecf, mask=m8)
        plsc.store_scatter(resy_v, [idx], vecy, mask=m8)


def _sc_worker(rpw, ncls, m, label_hbm, pred_hbm, outf_hbm, outy_hbm,
               labels_v, buf0, buf1, resf_v, resy_v, sem0, sem1):
    cid = lax.axis_index("c")
    sid = lax.axis_index("s")
    wid = sid * _SC_NC + cid
    base_row = wid * rpw
    mc = (m * _W) // _CW
    n_chunks = (rpw // 8) * mc

    def start(t, buf, sem):
        g = t // mc
        ch = t % mc
        pltpu.make_async_copy(
            pred_hbm.at[pl.ds(base_row + g * 8, 8), pl.ds(ch * _CW, _CW)],
            buf, sem).start()

    def wait(t, buf, sem):
        g = t // mc
        ch = t % mc
        pltpu.make_async_copy(
            pred_hbm.at[pl.ds(base_row + g * 8, 8), pl.ds(ch * _CW, _CW)],
            buf, sem).wait()

    pltpu.sync_copy(label_hbm.at[pl.ds(base_row, rpw)], labels_v)
    start(0, buf0, sem0)

    def pair_body(u, carry):
        accs, fys = list(carry[0]), list(carry[1])
        t0 = 2 * u
        start(t0 + 1, buf1, sem1)
        wait(t0, buf0, sem0)
        accs, fys = _sc_process_chunk(buf0, labels_v, t0, mc, accs, fys)

        @pl.when(t0 + 2 < n_chunks)
        def _next():
            start(t0 + 2, buf0, sem0)

        _sc_finalize_group(t0, mc, n_chunks, accs, fys, resf_v, resy_v)

        t1 = t0 + 1
        wait(t1, buf1, sem1)
        accs, fys = _sc_process_chunk(buf1, labels_v, t1, mc, accs, fys)
        _sc_finalize_group(t1, mc, n_chunks, accs, fys, resf_v, resy_v)
        return tuple(accs), tuple(fys)

    init = (tuple(jnp.full((16,), _NEG, jnp.float32) for _ in range(8)),
            tuple(jnp.float32(_NEG) for _ in range(8)))
    lax.fori_loop(0, n_chunks // 2, pair_body, init)

    pltpu.sync_copy(resf_v, outf_hbm.at[pl.ds(base_row, rpw)])
    pltpu.sync_copy(resy_v, outy_hbm.at[pl.ds(base_row, rpw)])


def _sc_loss(prediction, label, m):
    nrows, ncls = prediction.shape
    rpw = nrows // _SC_NW

    mesh = plsc.VectorSubcoreMesh(core_axis_name="c", subcore_axis_name="s")
    call = functools.partial(
        pl.kernel,
        out_type=(jax.ShapeDtypeStruct((nrows,), jnp.float32),
                  jax.ShapeDtypeStruct((nrows,), jnp.float32)),
        mesh=mesh,
        scratch_types=[
            pltpu.VMEM((rpw,), jnp.int32),
            pltpu.VMEM((8, _CW), jnp.float32),
            pltpu.VMEM((8, _CW), jnp.float32),
            pltpu.VMEM((rpw,), jnp.float32),
            pltpu.VMEM((rpw,), jnp.float32),
            pltpu.SemaphoreType.DMA,
            pltpu.SemaphoreType.DMA,
        ],
        compiler_params=pltpu.CompilerParams(needs_layout_passes=False),
    )(functools.partial(_sc_worker, rpw, ncls, m))

    return call(label, prediction)


def _lane_tree_max(v, width):
    while width > 128:
        width //= 2
        v = jnp.maximum(v[:, :width], v[:, width:2 * width])
    return v


def _tc_body(br, ncls, m, jn, label_ref, pred_ref, outf_ref, outy_ref,
             accm_ref, accy_ref):
    j = pl.program_id(0)

    @pl.when(j == 0)
    def _init():
        accm_ref[...] = jnp.full((br, 128), _NEG, jnp.float32)
        accy_ref[...] = jnp.full((br, 128), _NEG, jnp.float32)

    x = pred_ref[...]
    lab = label_ref[...]
    base = jax.lax.broadcasted_iota(jnp.int32, (br, _W), 1) + (m + j) * _W
    matched = base == lab
    invalid = base >= ncls
    xm = jnp.where(matched | invalid, -1e10, x)
    fyv = jnp.where(matched, x, _NEG)
    accm_ref[...] = jnp.maximum(accm_ref[...], _lane_tree_max(xm, _W))
    accy_ref[...] = jnp.maximum(accy_ref[...], _lane_tree_max(fyv, _W))

    @pl.when(j == jn - 1)
    def _fin():
        outf_ref[...] = jnp.max(accm_ref[...], axis=1, keepdims=True)
        outy_ref[...] = jnp.max(accy_ref[...], axis=1, keepdims=True)


def _tc_loss(prediction, label, m):
    nrows, ncls = prediction.shape
    br = nrows
    jn = pl.cdiv(ncls, _W) - m
    label2 = label.reshape(nrows, 1)

    body = functools.partial(_tc_body, br, ncls, m, jn)
    return pl.pallas_call(
        body,
        grid=(jn,),
        in_specs=[
            pl.BlockSpec((br, 1), lambda j: (0, 0)),
            pl.BlockSpec((br, _W), lambda j: (0, j + m)),
        ],
        out_specs=[
            pl.BlockSpec((br, 1), lambda j: (0, 0)),
            pl.BlockSpec((br, 1), lambda j: (0, 0)),
        ],
        out_shape=[
            jax.ShapeDtypeStruct((nrows, 1), jnp.float32),
            jax.ShapeDtypeStruct((nrows, 1), jnp.float32),
        ],
        scratch_shapes=[
            pltpu.VMEM((br, 128), jnp.float32),
            pltpu.VMEM((br, 128), jnp.float32),
        ],
        compiler_params=pltpu.CompilerParams(
            dimension_semantics=("arbitrary",)),
    )(label2, prediction)


def _combine_body(nrows, ftc_ref, ytc_ref, fsc_ref, ysc_ref, out_ref):
    fnym = jnp.maximum(ftc_ref[...], fsc_ref[...])
    fy = jnp.maximum(ytc_ref[...], ysc_ref[...])
    l = (jnp.maximum(_MARGIN_M + _MARGIN_T - fy, 0.0)
         + jnp.maximum(_MARGIN_M + fnym, 0.0))
    out_ref[0, 0] = jnp.sum(l) / nrows


def kernel(prediction, label):
    nrows, _ = prediction.shape
    fsc, ysc = _sc_loss(prediction, label, _SC_M)
    ftc, ytc = _tc_loss(prediction, label, _SC_M)

    shaped = [a.reshape(8, nrows // 8) for a in (ftc, ytc, fsc, ysc)]
    out = pl.pallas_call(
        functools.partial(_combine_body, nrows),
        out_specs=pl.BlockSpec(memory_space=pltpu.SMEM),
        out_shape=jax.ShapeDtypeStruct((1, 1), jnp.float32),
    )(*shaped)
    return out[0, 0]


# speedup vs baseline: 1.0675x; 1.0053x over previous
import functools

import jax
import jax.numpy as jnp
from jax import lax
from jax.experimental import pallas as pl
from jax.experimental.pallas import tpu as pltpu
from jax.experimental.pallas import tpu_sc as plsc

_MARGIN_M = 1.0
_MARGIN_T = 1.0

_W = 2048
_SC_M = 24
_CW = 4096

_SC_NC = 2
_SC_NS = 16
_SC_NW = _SC_NC * _SC_NS
_NEG = -3.4e38


def _sc_extract_lab(labels_v, g, rr):
    lslice = labels_v[pl.ds((g // 2) * 16, 16)]
    lane = (g % 2) * 8 + rr
    return jnp.max(jnp.where(lax.iota(jnp.int32, 16) == lane,
                             lslice.astype(jnp.float32), -1.0)
                   ).astype(jnp.int32)


def _sc_process_chunk(buf, labels_v, t, mc, accs, fys):
    g = t // mc
    ch = t % mc
    first = ch == 0
    col0 = ch * _CW

    accs = [jnp.where(first, jnp.full((16,), _NEG, jnp.float32), a)
            for a in accs]
    fys = [jnp.where(first, _NEG, f) for f in fys]

    lanes16 = lax.iota(jnp.int32, 16)
    for rr in range(8):
        lab = _sc_extract_lab(labels_v, g, rr)
        in_c = jnp.logical_and(lab >= col0, lab < col0 + _CW)
        off = jnp.where(in_c, lab - col0, 0)
        sbase = (off // 16) * 16
        lane = off - sbase
        sl = buf[rr, pl.ds(sbase, 16)]
        mask = jnp.logical_and(lanes16 == lane, in_c)
        fy_c = jnp.max(jnp.where(mask, sl, _NEG))
        fys[rr] = jnp.maximum(fys[rr], jnp.where(in_c, fy_c, _NEG))
        buf[rr, pl.ds(sbase, 16)] = jnp.where(mask, -1e10, sl)

    def step(i, carry):
        out = list(carry)
        base = i * 128
        for rr in range(8):
            for j in range(8):
                v = buf[rr, pl.ds(base + j * 16, 16)]
                out[rr] = jnp.maximum(out[rr], v)
        return tuple(out)

    accs = lax.fori_loop(0, _CW // 128, step, tuple(accs))
    return list(accs), fys


def _sc_finalize_group(t, mc, n_chunks, accs, fys, resf_v, resy_v):
    g = t // mc
    last = (t % mc) == mc - 1

    @pl.when(last)
    def _fin():
        lanes16 = lax.iota(jnp.int32, 16)
        vecf = jnp.full((16,), _NEG, jnp.float32)
        vecy = jnp.full((16,), _NEG, jnp.float32)
        for rr in range(8):
            vecf = jnp.where(lanes16 == rr, jnp.max(accs[rr]), vecf)
            vecy = jnp.where(lanes16 == rr, fys[rr], vecy)
        idx = lanes16 + g * 8
        m8 = lanes16 < 8
        plsc.store_scatter(resf_v, [idx], vecf, mask=m8)
        plsc.store_scatter(resy_v, [idx], vecy, mask=m8)


def _sc_worker(rpw, ncls, m, label_hbm, pred_hbm, outf_hbm, outy_hbm,
               labels_v, buf0, buf1, resf_v, resy_v, sem0, sem1):
    cid = lax.axis_index("c")
    sid = lax.axis_index("s")
    wid = sid * _SC_NC + cid
    base_row = wid * rpw
    mc = (m * _W) // _CW
    n_chunks = (rpw // 8) * mc

    def start(t, buf, sem):
        g = t // mc
        ch = t % mc
        pltpu.make_async_copy(
            pred_hbm.at[pl.ds(base_row + g * 8, 8), pl.ds(ch * _CW, _CW)],
            buf, sem).start()

    def wait(t, buf, sem):
        g = t // mc
        ch = t % mc
        pltpu.make_async_copy(
            pred_hbm.at[pl.ds(base_row + g * 8, 8), pl.ds(ch * _CW, _CW)],
            buf, sem).wait()

    pltpu.sync_copy(label_hbm.at[pl.ds(base_row, rpw)], labels_v)
    start(0, buf0, sem0)

    def pair_body(u, carry):
        accs, fys = list(carry[0]), list(carry[1])
        t0 = 2 * u
        start(t0 + 1, buf1, sem1)
        wait(t0, buf0, sem0)
        accs, fys = _sc_process_chunk(buf0, labels_v, t0, mc, accs, fys)

        @pl.when(t0 + 2 < n_chunks)
        def _next():
            start(t0 + 2, buf0, sem0)

        _sc_finalize_group(t0, mc, n_chunks, accs, fys, resf_v, resy_v)

        t1 = t0 + 1
        wait(t1, buf1, sem1)
        accs, fys = _sc_process_chunk(buf1, labels_v, t1, mc, accs, fys)
        _sc_finalize_group(t1, mc, n_chunks, accs, fys, resf_v, resy_v)
        return tuple(accs), tuple(fys)

    init = (tuple(jnp.full((16,), _NEG, jnp.float32) for _ in range(8)),
            tuple(jnp.float32(_NEG) for _ in range(8)))
    lax.fori_loop(0, n_chunks // 2, pair_body, init)

    pltpu.sync_copy(resf_v, outf_hbm.at[pl.ds(base_row, rpw)])
    pltpu.sync_copy(resy_v, outy_hbm.at[pl.ds(base_row, rpw)])


def _sc_loss(prediction, label, m):
    nrows, ncls = prediction.shape
    rpw = nrows // _SC_NW

    mesh = plsc.VectorSubcoreMesh(core_axis_name="c", subcore_axis_name="s")
    call = functools.partial(
        pl.kernel,
        out_type=(jax.ShapeDtypeStruct((nrows,), jnp.float32),
                  jax.ShapeDtypeStruct((nrows,), jnp.float32)),
        mesh=mesh,
        scratch_types=[
            pltpu.VMEM((rpw,), jnp.int32),
            pltpu.VMEM((8, _CW), jnp.float32),
            pltpu.VMEM((8, _CW), jnp.float32),
            pltpu.VMEM((rpw,), jnp.float32),
            pltpu.VMEM((rpw,), jnp.float32),
            pltpu.SemaphoreType.DMA,
            pltpu.SemaphoreType.DMA,
        ],
        compiler_params=pltpu.CompilerParams(needs_layout_passes=False),
    )(functools.partial(_sc_worker, rpw, ncls, m))

    return call(label, prediction)


def _lane_tree_max(v, width):
    while width > 128:
        width //= 2
        v = jnp.maximum(v[:, :width], v[:, width:2 * width])
    return v


def _tc_body(br, ncls, m, jn, label_ref, pred_ref, outf_ref, outy_ref,
             accm_ref, accy_ref):
    j = pl.program_id(0)

    @pl.when(j == 0)
    def _init():
        accm_ref[...] = jnp.full((br, 128), _NEG, jnp.float32)
        accy_ref[...] = jnp.full((br, 128), _NEG, jnp.float32)

    x = pred_ref[...]
    lab = label_ref[...]
    base = jax.lax.broadcasted_iota(jnp.int32, (br, _W), 1) + (m + j) * _W
    matched = base == lab
    invalid = base >= ncls
    xm = jnp.where(matched | invalid, -1e10, x)
    fyv = jnp.where(matched, x, _NEG)
    accm_ref[...] = jnp.maximum(accm_ref[...], _lane_tree_max(xm, _W))
    accy_ref[...] = jnp.maximum(accy_ref[...], _lane_tree_max(fyv, _W))

    @pl.when(j == jn - 1)
    def _fin():
        outf_ref[...] = jnp.max(accm_ref[...], axis=1, keepdims=True)
        outy_ref[...] = jnp.max(accy_ref[...], axis=1, keepdims=True)


def _tc_loss(prediction, label, m):
    nrows, ncls = prediction.shape
    br = nrows
    jn = pl.cdiv(ncls, _W) - m
    label2 = label.reshape(nrows, 1)

    body = functools.partial(_tc_body, br, ncls, m, jn)
    return pl.pallas_call(
        body,
        grid=(jn,),
        in_specs=[
            pl.BlockSpec((br, 1), lambda j: (0, 0)),
            pl.BlockSpec((br, _W), lambda j: (0, j + m)),
        ],
        out_specs=[
            pl.BlockSpec((br, 1), lambda j: (0, 0)),
            pl.BlockSpec((br, 1), lambda j: (0, 0)),
        ],
        out_shape=[
            jax.ShapeDtypeStruct((nrows, 1), jnp.float32),
            jax.ShapeDtypeStruct((nrows, 1), jnp.float32),
        ],
        scratch_shapes=[
            pltpu.VMEM((br, 128), jnp.float32),
            pltpu.VMEM((br, 128), jnp.float32),
        ],
        compiler_params=pltpu.CompilerParams(
            dimension_semantics=("arbitrary",)),
    )(label2, prediction)


def _combine_body(nrows, ftc_ref, ytc_ref, fsc_ref, ysc_ref, out_ref):
    fnym = jnp.maximum(ftc_ref[...], fsc_ref[...])
    fy = jnp.maximum(ytc_ref[...], ysc_ref[...])
    l = (jnp.maximum(_MARGIN_M + _MARGIN_T - fy, 0.0)
         + jnp.maximum(_MARGIN_M + fnym, 0.0))
    out_ref[0, 0] = jnp.sum(l) / nrows


def kernel(prediction, label):
    nrows, _ = prediction.shape
    fsc, ysc = _sc_loss(prediction, label, _SC_M)
    ftc, ytc = _tc_loss(prediction, label, _SC_M)

    shaped = [a.reshape(8, nrows // 8) for a in (ftc, ytc, fsc, ysc)]
    out = pl.pallas_call(
        functools.partial(_combine_body, nrows),
        out_specs=pl.BlockSpec(memory_space=pltpu.SMEM),
        out_shape=jax.ShapeDtypeStruct((1, 1), jnp.float32),
    )(*shaped)
    return out[0, 0]
